# baseline (speedup 1.0000x reference)
import jax
import jax.numpy as jnp
from jax.experimental import pallas as pl
from jax.experimental.pallas import tpu as pltpu

GROUPS = 2
NQ = 4
K = 1024
D = 256
B, T = 16, 576
ROWS = B * T
R = 1152
NT = ROWS // R


def _vq_body(x_ref, cb_ref, c2_ref, q_ref, idx_ref, loss_ref, ch_scr):
    i = pl.program_id(1)

    @pl.when(i == 0)
    def _init():
        loss_ref[...] = jnp.zeros_like(loss_ref)
        for q in range(NQ):
            u = jax.lax.bitcast_convert_type(cb_ref[0, q], jnp.uint32)
            ch_scr[q] = jnp.concatenate(
                [(((u >> s) & jnp.uint32(0xFF)).astype(jnp.int32) - 128)
                 .astype(jnp.int8) for s in (0, 8, 16, 24)], axis=1)

    x0 = x_ref[...]
    iota_f = jax.lax.broadcasted_iota(jnp.int32, (1, K), 1).astype(jnp.float32)
    qrow = jax.lax.broadcasted_iota(jnp.int32, (8, 128), 0)
    loss_acc = jnp.zeros((8, 128), jnp.float32)
    H = R // 2
    r = [x0[:H], x0[H:]]
    rsq = [r[0] * r[0], r[1] * r[1]]
    for q in range(NQ):
        cbq = cb_ref[0, q]
        c2row = c2_ref[0, q:q + 1, :]
        lqs = []
        for h in (0, 1):
            rowsq = jnp.sum(rsq[h], axis=1, keepdims=True)
            fc = jnp.dot(r[h], cbq.T, preferred_element_type=jnp.float32)
            dist = (rowsq - 2.0 * fc) + c2row
            min_d = jnp.min(dist, axis=1, keepdims=True)
            cand = jnp.where(dist == min_d, iota_f, float(K))
            idxf = jnp.min(cand, axis=1, keepdims=True)
            idx_ref[0, h * H:(h + 1) * H, q:q + 1] = idxf.astype(jnp.int32)
            onehot = (iota_f == idxf).astype(jnp.int8)
            ball = jnp.dot(onehot, ch_scr[q], preferred_element_type=jnp.int32)
            bits = ((ball[:, 3 * D:] << 24) + (ball[:, 2 * D:3 * D] << 16)
                    + (ball[:, D:2 * D] << 8) + ball[:, :D]
                    + jnp.int32(-2139062144))
            z = jax.lax.bitcast_convert_type(bits, jnp.float32)
            r[h] = r[h] - z
            rsq[h] = r[h] * r[h]
            lqs.append(jnp.sum(rsq[h]))
        loss_acc = loss_acc + jnp.where(qrow == q, lqs[0] + lqs[1], 0.0)
    q_ref[:H] = x0[:H] - r[0]
    q_ref[H:] = x0[H:] - r[1]
    loss_ref[0] += loss_acc


def kernel(x, codebooks):
    xf = x.reshape(ROWS, GROUPS * D)
    c2 = jnp.sum(codebooks * codebooks, axis=-1)
    c2p = jnp.concatenate(
        [c2, jnp.zeros((GROUPS, 8 - NQ, K), jnp.float32)], axis=1)

    grid = (GROUPS, NT)
    qflat, idx_out, loss_out = pl.pallas_call(
        _vq_body,
        grid=grid,
        in_specs=[
            pl.BlockSpec((R, D), lambda g, i: (i, g)),
            pl.BlockSpec((1, NQ, K, D), lambda g, i: (g, 0, 0, 0)),
            pl.BlockSpec((1, 8, K), lambda g, i: (g, 0, 0)),
        ],
        out_specs=[
            pl.BlockSpec((R, D), lambda g, i: (i, g)),
            pl.BlockSpec((1, R, 8), lambda g, i: (g, i, 0)),
            pl.BlockSpec((1, 8, 128), lambda g, i: (g, 0, 0)),
        ],
        out_shape=[
            jax.ShapeDtypeStruct((ROWS, GROUPS * D), jnp.float32),
            jax.ShapeDtypeStruct((GROUPS, ROWS, 8), jnp.int32),
            jax.ShapeDtypeStruct((GROUPS, 8, 128), jnp.float32),
        ],
        scratch_shapes=[pltpu.VMEM((NQ, K, 4 * D), jnp.int8)],
        compiler_params=pltpu.CompilerParams(
            dimension_semantics=("parallel", "arbitrary")),
    )(xf, codebooks, c2p)

    quantized = qflat.reshape(B, T, GROUPS * D)
    all_indices = jnp.transpose(idx_out, (0, 2, 1))[:, :NQ].reshape(
        GROUPS, NQ, B, T)
    commit_losses = 1.25 * jnp.sum(loss_out[:, :NQ, 0], axis=1) / (ROWS * D)
    return quantized, all_indices, commit_losses


# speedup vs baseline: 1.0358x; 1.0358x over previous
import jax
import jax.numpy as jnp
from jax.experimental import pallas as pl
from jax.experimental.pallas import tpu as pltpu

GROUPS = 2
NQ = 4
K = 1024
D = 256
B, T = 16, 576
ROWS = B * T
R = 1536
NT = ROWS // R


def _vq_body(x_ref, cb_ref, c2_ref, q_ref, idx_ref, loss_ref, ch_scr):
    i = pl.program_id(1)

    @pl.when(i == 0)
    def _init():
        loss_ref[...] = jnp.zeros_like(loss_ref)
        for q in range(NQ):
            u = jax.lax.bitcast_convert_type(cb_ref[0, q], jnp.uint32)
            ch_scr[q] = jnp.concatenate(
                [(((u >> s) & jnp.uint32(0xFF)).astype(jnp.int32) - 128)
                 .astype(jnp.int8) for s in (0, 8, 16, 24)], axis=1)

    x0 = x_ref[...]
    r = x0
    iota_f = jax.lax.broadcasted_iota(jnp.int32, (1, K), 1).astype(jnp.float32)
    qrow = jax.lax.broadcasted_iota(jnp.int32, (8, 128), 0)
    loss_acc = jnp.zeros((8, 128), jnp.float32)
    rsq = r * r
    for q in range(NQ):
        cbq = cb_ref[0, q]
        c2row = c2_ref[0, q:q + 1, :]
        rowsq = jnp.sum(rsq, axis=1, keepdims=True)
        fc = jnp.dot(r, cbq.T, preferred_element_type=jnp.float32)
        dist = (rowsq - 2.0 * fc) + c2row
        min_d = jnp.min(dist, axis=1, keepdims=True)
        cand = jnp.where(dist == min_d, iota_f, float(K))
        idxf = jnp.min(cand, axis=1, keepdims=True)
        idx_ref[0, :, q:q + 1] = idxf.astype(jnp.int32)
        onehot = (iota_f == idxf).astype(jnp.int8)
        ball = jnp.dot(onehot, ch_scr[q], preferred_element_type=jnp.int32)
        bits = ((ball[:, 3 * D:] << 24) + (ball[:, 2 * D:3 * D] << 16)
                + (ball[:, D:2 * D] << 8) + ball[:, :D]
                + jnp.int32(-2139062144))
        z = jax.lax.bitcast_convert_type(bits, jnp.float32)
        r = r - z
        rsq = r * r
        lq = jnp.sum(rsq)
        loss_acc = loss_acc + jnp.where(qrow == q, lq, 0.0)
    q_ref[...] = x0 - r
    loss_ref[0] += loss_acc


def kernel(x, codebooks):
    xf = x.reshape(ROWS, GROUPS * D)
    c2 = jnp.sum(codebooks * codebooks, axis=-1)
    c2p = jnp.concatenate(
        [c2, jnp.zeros((GROUPS, 8 - NQ, K), jnp.float32)], axis=1)

    grid = (GROUPS, NT)
    qflat, idx_out, loss_out = pl.pallas_call(
        _vq_body,
        grid=grid,
        in_specs=[
            pl.BlockSpec((R, D), lambda g, i: (i, g)),
            pl.BlockSpec((1, NQ, K, D), lambda g, i: (g, 0, 0, 0)),
            pl.BlockSpec((1, 8, K), lambda g, i: (g, 0, 0)),
        ],
        out_specs=[
            pl.BlockSpec((R, D), lambda g, i: (i, g)),
            pl.BlockSpec((1, R, 8), lambda g, i: (g, i, 0)),
            pl.BlockSpec((1, 8, 128), lambda g, i: (g, 0, 0)),
        ],
        out_shape=[
            jax.ShapeDtypeStruct((ROWS, GROUPS * D), jnp.float32),
            jax.ShapeDtypeStruct((GROUPS, ROWS, 8), jnp.int32),
            jax.ShapeDtypeStruct((GROUPS, 8, 128), jnp.float32),
        ],
        scratch_shapes=[pltpu.VMEM((NQ, K, 4 * D), jnp.int8)],
        compiler_params=pltpu.CompilerParams(
            dimension_semantics=("parallel", "arbitrary")),
    )(xf, codebooks, c2p)

    quantized = qflat.reshape(B, T, GROUPS * D)
    all_indices = jnp.transpose(idx_out, (0, 2, 1))[:, :NQ].reshape(
        GROUPS, NQ, B, T)
    commit_losses = 1.25 * jnp.sum(loss_out[:, :NQ, 0], axis=1) / (ROWS * D)
    return quantized, all_indices, commit_losses


# speedup vs baseline: 1.0495x; 1.0132x over previous
import jax
import jax.numpy as jnp
from jax.experimental import pallas as pl
from jax.experimental.pallas import tpu as pltpu

GROUPS = 2
NQ = 4
K = 1024
D = 256
B, T = 16, 576
ROWS = B * T
R = 1152
NT = ROWS // R


def _vq_body(x_ref, cb_ref, c2_ref, q_ref, idx_ref, loss_ref, ch_scr):
    i = pl.program_id(1)

    @pl.when(i == 0)
    def _init():
        loss_ref[...] = jnp.zeros_like(loss_ref)
        for q in range(NQ):
            u = jax.lax.bitcast_convert_type(cb_ref[0, q] * -0.5, jnp.uint32)
            ch_scr[q] = jnp.concatenate(
                [(((u >> s) & jnp.uint32(0xFF)).astype(jnp.int32) - 128)
                 .astype(jnp.int8) for s in (0, 8, 16, 24)], axis=1)

    x0 = x_ref[...]
    r = x0
    iota_f = jax.lax.broadcasted_iota(jnp.int32, (1, K), 1).astype(jnp.float32)
    qrow = jax.lax.broadcasted_iota(jnp.int32, (8, 128), 0)
    loss_acc = jnp.zeros((8, 128), jnp.float32)
    rowsq = jnp.sum(r * r, axis=1, keepdims=True)
    for q in range(NQ):
        cbq = cb_ref[0, q]
        c2row = c2_ref[0, q:q + 1, :]
        fc = jnp.dot(r, cbq.T, preferred_element_type=jnp.float32)
        dist = (rowsq + fc) + c2row
        min_d = jnp.min(dist, axis=1, keepdims=True)
        cand = jnp.where(dist == min_d, iota_f, float(K))
        idxf = jnp.min(cand, axis=1, keepdims=True)
        idx_ref[0, :, q:q + 1] = idxf.astype(jnp.int32)
        onehot = (iota_f == idxf).astype(jnp.int8)
        ball = jnp.dot(onehot, ch_scr[q], preferred_element_type=jnp.int32)
        bits = ((ball[:, 3 * D:] << 24) + (ball[:, 2 * D:3 * D] << 16)
                + (ball[:, D:2 * D] << 8) + ball[:, :D]
                + jnp.int32(-2139062144))
        z = jax.lax.bitcast_convert_type(bits, jnp.float32)
        r = r - z
        rowsq = jnp.sum(r * r, axis=1, keepdims=True)
        lq = jnp.sum(rowsq)
        loss_acc = loss_acc + jnp.where(qrow == q, lq, 0.0)
    q_ref[...] = x0 - r
    loss_ref[0] += loss_acc


def kernel(x, codebooks):
    xf = x.reshape(ROWS, GROUPS * D)
    cbn2 = codebooks * -2.0
    c2 = jnp.sum(codebooks * codebooks, axis=-1)
    c2p = jnp.concatenate(
        [c2, jnp.zeros((GROUPS, 8 - NQ, K), jnp.float32)], axis=1)

    grid = (GROUPS, NT)
    qflat, idx_out, loss_out = pl.pallas_call(
        _vq_body,
        grid=grid,
        in_specs=[
            pl.BlockSpec((R, D), lambda g, i: (i, g)),
            pl.BlockSpec((1, NQ, K, D), lambda g, i: (g, 0, 0, 0)),
            pl.BlockSpec((1, 8, K), lambda g, i: (g, 0, 0)),
        ],
        out_specs=[
            pl.BlockSpec((R, D), lambda g, i: (i, g)),
            pl.BlockSpec((1, R, 8), lambda g, i: (g, i, 0)),
            pl.BlockSpec((1, 8, 128), lambda g, i: (g, 0, 0)),
        ],
        out_shape=[
            jax.ShapeDtypeStruct((ROWS, GROUPS * D), jnp.float32),
            jax.ShapeDtypeStruct((GROUPS, ROWS, 8), jnp.int32),
            jax.ShapeDtypeStruct((GROUPS, 8, 128), jnp.float32),
        ],
        scratch_shapes=[pltpu.VMEM((NQ, K, 4 * D), jnp.int8)],
        compiler_params=pltpu.CompilerParams(
            dimension_semantics=("parallel", "arbitrary")),
    )(xf, cbn2, c2p)

    quantized = qflat.reshape(B, T, GROUPS * D)
    all_indices = jnp.transpose(idx_out, (0, 2, 1))[:, :NQ].reshape(
        GROUPS, NQ, B, T)
    commit_losses = 1.25 * jnp.sum(loss_out[:, :NQ, 0], axis=1) / (ROWS * D)
    return quantized, all_indices, commit_losses
